# baseline (speedup 1.0000x reference)
import jax
import jax.numpy as jnp
from jax import lax
from jax.experimental import pallas as pl
from jax.experimental.pallas import tpu as pltpu
from jax.experimental.pallas import tpu_sc as plsc

_B, _CTX, _D, _V = 4096, 20, 64, 100000
_DA = 80

_NC, _NS = 2, 16
_NW = _NC * _NS
_BPW = _B // _NW
_EC = 32
_NCHUNK = _BPW // _EC
_GE = 4
_GPC = _EC // _GE
_ROWS_G = _GE * _CTX
_ROWS_C = _EC * _CTX


def _pool_body(xf_hbm, tab_hbm, out_hbm, idx_v, buf0, buf1, out_v, sem0, sem1):
    wid = lax.axis_index("s") * _NC + lax.axis_index("c")
    base = wid * _BPW
    pltpu.sync_copy(xf_hbm.at[pl.ds(base * _CTX, _BPW * _CTX)], idx_v)

    bufs = (buf0, buf1)
    sems = (sem0, sem1)
    lane = lax.broadcasted_iota(jnp.int32, (16,), 0)
    ones_col = jnp.where(lane == 0, 1.0, 0.0).astype(jnp.float32)

    def fire(t, buf, sem):
        for g in range(_GPC):
            off = t * _ROWS_C + g * _ROWS_G
            pltpu.async_copy(
                tab_hbm.at[idx_v.at[pl.ds(off, _ROWS_G)]],
                buf.at[pl.ds(g * _ROWS_G, _ROWS_G)],
                sem,
            )

    def drain(buf, sem):
        pltpu.make_async_copy(tab_hbm.at[pl.ds(0, _ROWS_C)], buf, sem).wait()

    def process(t, buf):
        def elem_body(e, _):
            rbase = e * _CTX
            for c in range(_D // 16):
                sl = pl.ds(c * 16, 16)

                def row_body(j, acc):
                    return acc + buf[rbase + j, sl]

                s = lax.fori_loop(0, _CTX, row_body,
                                  jnp.zeros((16,), jnp.float32))
                out_v[t * _EC + e, sl] = s * (1.0 / _CTX)
            out_v[t * _EC + e, pl.ds(_D, 16)] = ones_col
            return 0

        lax.fori_loop(0, _EC, elem_body, 0)

    fire(0, bufs[0], sems[0])
    for t in range(_NCHUNK):
        if t + 1 < _NCHUNK:
            fire(t + 1, bufs[(t + 1) % 2], sems[(t + 1) % 2])
        drain(bufs[t % 2], sems[t % 2])
        process(t, bufs[t % 2])

    pltpu.sync_copy(out_v, out_hbm.at[pl.ds(base, _BPW)])


def _pool(x_flat, emb_table):
    return pl.kernel(
        _pool_body,
        out_type=jax.ShapeDtypeStruct((_B, _DA), jnp.float32),
        mesh=plsc.VectorSubcoreMesh(core_axis_name="c", subcore_axis_name="s"),
        scratch_types=[
            pltpu.VMEM((_BPW * _CTX,), jnp.int32),
            pltpu.VMEM((_ROWS_C, _D), jnp.float32),
            pltpu.VMEM((_ROWS_C, _D), jnp.float32),
            pltpu.VMEM((_BPW, _DA), jnp.float32),
            pltpu.SemaphoreType.DMA,
            pltpu.SemaphoreType.DMA,
        ],
        compiler_params=pltpu.CompilerParams(use_tc_tiling_on_sc=False),
    )(x_flat, emb_table)


_VB = 200
_U = 10
_VSTEP = _VB * _U
_NSTEP = _V // _VSTEP


def _mm_body(w_ref, p_ref, o_hbm, *slots):
    scratches = slots[:_U]
    sems = slots[_U:]
    i = pl.program_id(0)

    for u in range(_U):
        @pl.when(i > 0)
        def _():
            pltpu.make_async_copy(
                scratches[u],
                o_hbm.at[pl.ds((i - 1) * _VSTEP + u * _VB, _VB), :],
                sems[u],
            ).wait()

        scratches[u][...] = lax.dot_general(
            w_ref[pl.ds(u * _VB, _VB), :], p_ref[...],
            dimension_numbers=(((1,), (1,)), ((), ())),
            preferred_element_type=jnp.float32,
        )
        pltpu.make_async_copy(
            scratches[u],
            o_hbm.at[pl.ds(i * _VSTEP + u * _VB, _VB), :],
            sems[u],
        ).start(priority=u % 2)

    @pl.when(i == _NSTEP - 1)
    def _():
        for u in range(_U):
            pltpu.make_async_copy(
                scratches[u],
                o_hbm.at[pl.ds(i * _VSTEP + u * _VB, _VB), :],
                sems[u],
            ).wait()


def _project_t(W_aug, pooled_aug):
    return pl.pallas_call(
        _mm_body,
        grid=(_NSTEP,),
        in_specs=[
            pl.BlockSpec((_VSTEP, _DA), lambda i: (i, 0)),
            pl.BlockSpec((_B, _DA), lambda i: (0, 0)),
        ],
        out_specs=pl.BlockSpec(memory_space=pl.ANY),
        out_shape=jax.ShapeDtypeStruct((_V, _B), jnp.float32),
        scratch_shapes=(
            [pltpu.VMEM((_VB, _B), jnp.float32) for _ in range(_U)]
            + [pltpu.SemaphoreType.DMA for _ in range(_U)]
        ),
    )(W_aug, pooled_aug)


def kernel(x, emb_table, W, b):
    pooled_aug = _pool(x.reshape(-1), emb_table)
    W_aug = jnp.concatenate(
        [W, b[:, None], jnp.zeros((_V, _DA - _D - 1), jnp.float32)], axis=1)
    out_t = _project_t(W_aug, pooled_aug)
    return out_t.T
